# baseline (speedup 1.0000x reference)
import dataclasses
import functools

import jax
import jax.numpy as jnp
from jax import lax
from jax.experimental import pallas as pl
from jax.experimental.pallas import tpu as pltpu
from jax.experimental.pallas import tpu_sc as plsc

C_IN = 64
C_OUT = 64
TILE = 256
L = 16


def _sc_compiler_params():
    cp = pltpu.CompilerParams()
    if "needs_layout_passes" in pltpu.CompilerParams.__dataclass_fields__:
        cp = dataclasses.replace(cp, needs_layout_passes=False)
    return cp


def _mlp_matmul_kernel(locsT_ref, w0t_ref, w1t_ref, w2t_ref, x_ref, y_ref):
    h = jnp.sin(jnp.dot(w0t_ref[...], locsT_ref[...], preferred_element_type=jnp.float32))
    h = jnp.sin(jnp.dot(w1t_ref[...], h, preferred_element_type=jnp.float32))
    gT = jnp.dot(w2t_ref[...].astype(jnp.bfloat16), h.astype(jnp.bfloat16),
                 preferred_element_type=jnp.float32)
    gr = gT.astype(jnp.bfloat16).reshape(C_IN, C_OUT, TILE)
    x = x_ref[...].astype(jnp.bfloat16)
    y = jax.lax.dot_general(
        x, gr,
        dimension_numbers=(((2,), (0,)), ((1,), (2,))),
        preferred_element_type=jnp.float32,
    )
    y_ref[...] = y.transpose(1, 0, 2)


def _sc_gather(feat_flat, idx1p, b, n_in, nnz, np64):
    nv = idx1p.shape[0]
    nch = nv // L
    mesh = plsc.VectorSubcoreMesh(core_axis_name="c", subcore_axis_name="s")

    @functools.partial(
        pl.kernel,
        out_type=jax.ShapeDtypeStruct((b * np64,), jnp.float32),
        mesh=mesh,
        scratch_types=[
            pltpu.VMEM((nv,), jnp.int32),
            pltpu.VMEM((2 * n_in,), jnp.float32),
            pltpu.VMEM((2 * n_in,), jnp.float32),
            pltpu.VMEM((2 * nv,), jnp.float32),
            pltpu.VMEM((2 * nv,), jnp.float32),
            pltpu.SemaphoreType.DMA((4,)),
        ],
        compiler_params=_sc_compiler_params(),
    )
    def gather_kernel(feat_hbm, idx_hbm, x_hbm, idx_v, src0, src1, dst0, dst1,
                      sems):
        w = lax.axis_index("s") * 2 + lax.axis_index("c")
        pltpu.sync_copy(idx_hbm, idx_v)
        iota16 = lax.iota(jnp.int32, L)
        srcs = (src0, src1)
        dsts = (dst0, dst1)

        def feat_slice(bi):
            return feat_hbm.at[pl.ds((bi * C_IN + 2 * w) * n_in, 2 * n_in)]

        def out_slice(bi):
            return x_hbm.at[pl.ds(bi * np64 + w * 2 * nnz, 2 * nnz)]

        pltpu.async_copy(feat_slice(0), src0, sems.at[0])

        def process(bi, par):
            src_v = srcs[par]
            dst_v = dsts[par]
            pltpu.make_async_copy(feat_slice(bi), src_v, sems.at[par]).wait()

            @pl.when(bi + 1 < b)
            def _():
                pltpu.async_copy(feat_slice(bi + 1), srcs[1 - par],
                                 sems.at[1 - par])

            @pl.when(bi >= 2)
            def _():
                pltpu.make_async_copy(
                    dst_v.at[pl.ds(0, 2 * nnz)],
                    out_slice(bi - 2), sems.at[2 + par],
                ).wait()

            @plsc.parallel_loop(0, nch, unroll=8)
            def _ch0(i):
                idx = idx_v[pl.ds(i * L, L)]
                v0 = plsc.load_gather(src_v, [idx])
                dst_v[pl.ds(i * L, L)] = v0

            @plsc.parallel_loop(0, nch, unroll=8)
            def _ch1(i):
                idx = idx_v[pl.ds(i * L, L)]
                v1 = plsc.load_gather(src_v, [idx + n_in])
                plsc.store_scatter(dst_v, [iota16 + (nnz + i * L)], v1)

            pltpu.async_copy(
                dst_v.at[pl.ds(0, 2 * nnz)],
                out_slice(bi), sems.at[2 + par],
            )

        @pl.loop(0, b // 2)
        def _batch(k):
            process(2 * k, 0)
            process(2 * k + 1, 1)

        pltpu.make_async_copy(
            dst0.at[pl.ds(0, 2 * nnz)], out_slice(b - 2), sems.at[2]
        ).wait()
        pltpu.make_async_copy(
            dst1.at[pl.ds(0, 2 * nnz)], out_slice(b - 1), sems.at[3]
        ).wait()

    return gather_kernel(feat_flat, idx1p)


def _sc_scatter(y_flat, idx0p, b, nnz, np64, n_out):
    nv = idx0p.shape[0]
    nch = nv // L
    acc_w = n_out + L
    mesh = plsc.VectorSubcoreMesh(core_axis_name="c", subcore_axis_name="s")

    @functools.partial(
        pl.kernel,
        out_type=jax.ShapeDtypeStruct((b * C_OUT * n_out,), jnp.float32),
        mesh=mesh,
        scratch_types=[
            pltpu.VMEM((nv,), jnp.int32),
            pltpu.VMEM((2 * nv,), jnp.float32),
            pltpu.VMEM((2 * nv,), jnp.float32),
            pltpu.VMEM((acc_w,), jnp.float32),
            pltpu.VMEM((n_out,), jnp.float32),
            pltpu.VMEM((n_out,), jnp.float32),
            pltpu.SemaphoreType.DMA((2,)),
            pltpu.SemaphoreType.DMA((2,)),
        ],
        compiler_params=_sc_compiler_params(),
    )
    def scatter_kernel(y_hbm, idx_hbm, out_hbm, idx_v, val0, val1, acc,
                       obuf0, obuf1, sems_in, sems_out):
        w = lax.axis_index("s") * 2 + lax.axis_index("c")
        pltpu.sync_copy(idx_hbm, idx_v)
        zeros16f = jnp.zeros((L,), jnp.float32)
        iota16 = lax.iota(jnp.int32, L)
        vals = (val0, val1)
        obufs = (obuf0, obuf1)

        def y_slice(bi):
            return y_hbm.at[pl.ds(bi * np64 + w * 2 * nnz, 2 * nnz)]

        def out_slice(bi, ch):
            return out_hbm.at[pl.ds((bi * C_OUT + 2 * w + ch) * n_out, n_out)]

        @plsc.parallel_loop(0, acc_w // L, unroll=4)
        def _zero(z):
            acc[pl.ds(z * L, L)] = zeros16f

        pltpu.async_copy(y_slice(0), val0.at[pl.ds(0, 2 * nnz)], sems_in.at[0])

        def process(bi, par):
            val_v = vals[par]
            pltpu.make_async_copy(y_slice(bi), val_v.at[pl.ds(0, 2 * nnz)],
                                  sems_in.at[par]).wait()

            @pl.when(bi + 1 < b)
            def _():
                pltpu.async_copy(y_slice(bi + 1),
                                 vals[1 - par].at[pl.ds(0, 2 * nnz)],
                                 sems_in.at[1 - par])

            for ch in range(2):
                if ch == 0:
                    @plsc.parallel_loop(0, nch, unroll=8)
                    def _acc0(i):
                        p = idx_v[pl.ds(i * L, L)]
                        v = val_v[pl.ds(i * L, L)]
                        plsc.addupdate_scatter(acc, [p], v)
                else:
                    @plsc.parallel_loop(0, nch, unroll=8)
                    def _acc1(i):
                        p = idx_v[pl.ds(i * L, L)]
                        v = plsc.load_gather(val_v, [iota16 + (nnz + i * L)])
                        plsc.addupdate_scatter(acc, [p], v)

                @pl.when(bi >= 1)
                def _():
                    pltpu.make_async_copy(
                        obufs[ch], out_slice(bi - 1, ch), sems_out.at[ch]
                    ).wait()

                @plsc.parallel_loop(0, n_out // L, unroll=4)
                def _drain(j):
                    s = acc[pl.ds(j * L, L)]
                    acc[pl.ds(j * L, L)] = zeros16f
                    obufs[ch][pl.ds(j * L, L)] = s

                pltpu.async_copy(obufs[ch], out_slice(bi, ch),
                                 sems_out.at[ch])

        @pl.loop(0, b // 2)
        def _batch(k):
            process(2 * k, 0)
            process(2 * k + 1, 1)

        pltpu.make_async_copy(obuf0, out_slice(b - 1, 0),
                              sems_out.at[0]).wait()
        pltpu.make_async_copy(obuf1, out_slice(b - 1, 1),
                              sems_out.at[1]).wait()

    return scatter_kernel(y_flat, idx0p)


def kernel(features, eval_locs, W0, W1, W2, eval_indices):
    b, c_in, n_in = features.shape
    nnz = eval_indices.shape[0]
    np_pad = ((nnz + TILE - 1) // TILE) * TILE
    nv = ((nnz + L - 1) // L) * L
    n_out = 1024

    idx0 = eval_indices[:, 0].astype(jnp.int32)
    idx1 = eval_indices[:, 1].astype(jnp.int32)
    idx1p = jnp.pad(idx1, (0, nv - nnz))
    idx0p = jnp.pad(idx0, (0, nv - nnz), constant_values=n_out)

    locsT_pad = jnp.pad(eval_locs, ((0, np_pad - nnz), (0, 0))).T

    def tc_stage(x3, bh):
        return pl.pallas_call(
            _mlp_matmul_kernel,
            grid=(np_pad // TILE,),
            in_specs=[
                pl.BlockSpec((2, TILE), lambda i: (0, i)),
                pl.BlockSpec((64, 2), lambda i: (0, 0)),
                pl.BlockSpec((64, 64), lambda i: (0, 0)),
                pl.BlockSpec((C_IN * C_OUT, 64), lambda i: (0, 0)),
                pl.BlockSpec((bh, TILE, C_IN), lambda i: (0, i, 0)),
            ],
            out_specs=pl.BlockSpec((bh, TILE, C_OUT), lambda i: (0, i, 0)),
            out_shape=jax.ShapeDtypeStruct((bh, np_pad, C_OUT), jnp.float32),
        )(locsT_pad, W0.T, W1.T, W2.T, x3)

    bh = b // 2
    outs = []
    for h in range(2):
        feat_h = features[h * bh:(h + 1) * bh].reshape(-1)
        x_flat = _sc_gather(feat_h, idx1p, bh, n_in, nnz, np_pad * C_IN)
        x3 = x_flat.reshape(bh, np_pad, C_IN)
        y3 = tc_stage(x3, bh)
        out_flat = _sc_scatter(y3.reshape(-1), idx0p, bh, nnz,
                               np_pad * C_OUT, n_out)
        outs.append(out_flat.reshape(bh, C_OUT, n_out))
    return jnp.concatenate(outs, axis=0)


# speedup vs baseline: 1.1666x; 1.1666x over previous
import dataclasses
import functools

import jax
import jax.numpy as jnp
from jax import lax
from jax.experimental import pallas as pl
from jax.experimental.pallas import tpu as pltpu
from jax.experimental.pallas import tpu_sc as plsc

C_IN = 64
C_OUT = 64
TILE = 512
L = 16


def _sc_compiler_params():
    cp = pltpu.CompilerParams()
    if "needs_layout_passes" in pltpu.CompilerParams.__dataclass_fields__:
        cp = dataclasses.replace(cp, needs_layout_passes=False)
    return cp


def _mlp_matmul_kernel(locsT_ref, w0t_ref, w1t_ref, w2t_ref, x_ref, y_ref):
    h = jnp.sin(jnp.dot(w0t_ref[...], locsT_ref[...], preferred_element_type=jnp.float32))
    h = jnp.sin(jnp.dot(w1t_ref[...], h, preferred_element_type=jnp.float32))
    gT = jnp.dot(w2t_ref[...].astype(jnp.bfloat16), h.astype(jnp.bfloat16),
                 preferred_element_type=jnp.float32)
    gr = gT.astype(jnp.bfloat16).reshape(C_IN, C_OUT, TILE)
    x = x_ref[...].astype(jnp.bfloat16)
    y = jax.lax.dot_general(
        x, gr,
        dimension_numbers=(((2,), (0,)), ((1,), (2,))),
        preferred_element_type=jnp.float32,
    )
    y_ref[...] = y.transpose(1, 0, 2)


def _sc_gather(feat_flat, idx1p, b, n_in, nnz, np64):
    nv = idx1p.shape[0]
    nch = nv // L
    mesh = plsc.VectorSubcoreMesh(core_axis_name="c", subcore_axis_name="s")

    @functools.partial(
        pl.kernel,
        out_type=jax.ShapeDtypeStruct((b * np64,), jnp.float32),
        mesh=mesh,
        scratch_types=[
            pltpu.VMEM((nv,), jnp.int32),
            pltpu.VMEM((2 * n_in,), jnp.float32),
            pltpu.VMEM((2 * n_in,), jnp.float32),
            pltpu.VMEM((2 * nv,), jnp.float32),
            pltpu.VMEM((2 * nv,), jnp.float32),
            pltpu.SemaphoreType.DMA((4,)),
        ],
        compiler_params=_sc_compiler_params(),
    )
    def gather_kernel(feat_hbm, idx_hbm, x_hbm, idx_v, src0, src1, dst0, dst1,
                      sems):
        w = lax.axis_index("s") * 2 + lax.axis_index("c")
        pltpu.sync_copy(idx_hbm, idx_v)
        iota16 = lax.iota(jnp.int32, L)
        srcs = (src0, src1)
        dsts = (dst0, dst1)

        def feat_slice(bi):
            return feat_hbm.at[pl.ds((bi * C_IN + 2 * w) * n_in, 2 * n_in)]

        def out_slice(bi):
            return x_hbm.at[pl.ds(bi * np64 + w * 2 * nnz, 2 * nnz)]

        pltpu.async_copy(feat_slice(0), src0, sems.at[0])

        def process(bi, par):
            src_v = srcs[par]
            dst_v = dsts[par]
            pltpu.make_async_copy(feat_slice(bi), src_v, sems.at[par]).wait()

            @pl.when(bi + 1 < b)
            def _():
                pltpu.async_copy(feat_slice(bi + 1), srcs[1 - par],
                                 sems.at[1 - par])

            @pl.when(bi >= 2)
            def _():
                pltpu.make_async_copy(
                    dst_v.at[pl.ds(0, 2 * nnz)],
                    out_slice(bi - 2), sems.at[2 + par],
                ).wait()

            @plsc.parallel_loop(0, nch, unroll=8)
            def _ch0(i):
                idx = idx_v[pl.ds(i * L, L)]
                v0 = plsc.load_gather(src_v, [idx])
                dst_v[pl.ds(i * L, L)] = v0

            @plsc.parallel_loop(0, nch, unroll=8)
            def _ch1(i):
                idx = idx_v[pl.ds(i * L, L)]
                v1 = plsc.load_gather(src_v, [idx + n_in])
                plsc.store_scatter(dst_v, [iota16 + (nnz + i * L)], v1)

            pltpu.async_copy(
                dst_v.at[pl.ds(0, 2 * nnz)],
                out_slice(bi), sems.at[2 + par],
            )

        @pl.loop(0, b // 2)
        def _batch(k):
            process(2 * k, 0)
            process(2 * k + 1, 1)

        pltpu.make_async_copy(
            dst0.at[pl.ds(0, 2 * nnz)], out_slice(b - 2), sems.at[2]
        ).wait()
        pltpu.make_async_copy(
            dst1.at[pl.ds(0, 2 * nnz)], out_slice(b - 1), sems.at[3]
        ).wait()

    return gather_kernel(feat_flat, idx1p)


def _sc_scatter(y_flat, idx0p, b, nnz, np64, n_out):
    nv = idx0p.shape[0]
    nch = nv // L
    acc_w = n_out + L
    mesh = plsc.VectorSubcoreMesh(core_axis_name="c", subcore_axis_name="s")

    @functools.partial(
        pl.kernel,
        out_type=jax.ShapeDtypeStruct((b * C_OUT * n_out,), jnp.float32),
        mesh=mesh,
        scratch_types=[
            pltpu.VMEM((nv,), jnp.int32),
            pltpu.VMEM((2 * nv,), jnp.float32),
            pltpu.VMEM((2 * nv,), jnp.float32),
            pltpu.VMEM((acc_w,), jnp.float32),
            pltpu.VMEM((n_out,), jnp.float32),
            pltpu.VMEM((n_out,), jnp.float32),
            pltpu.SemaphoreType.DMA((2,)),
            pltpu.SemaphoreType.DMA((2,)),
        ],
        compiler_params=_sc_compiler_params(),
    )
    def scatter_kernel(y_hbm, idx_hbm, out_hbm, idx_v, val0, val1, acc,
                       obuf0, obuf1, sems_in, sems_out):
        w = lax.axis_index("s") * 2 + lax.axis_index("c")
        pltpu.sync_copy(idx_hbm, idx_v)
        zeros16f = jnp.zeros((L,), jnp.float32)
        iota16 = lax.iota(jnp.int32, L)
        vals = (val0, val1)
        obufs = (obuf0, obuf1)

        def y_slice(bi):
            return y_hbm.at[pl.ds(bi * np64 + w * 2 * nnz, 2 * nnz)]

        def out_slice(bi, ch):
            return out_hbm.at[pl.ds((bi * C_OUT + 2 * w + ch) * n_out, n_out)]

        @plsc.parallel_loop(0, acc_w // L, unroll=4)
        def _zero(z):
            acc[pl.ds(z * L, L)] = zeros16f

        pltpu.async_copy(y_slice(0), val0.at[pl.ds(0, 2 * nnz)], sems_in.at[0])

        def process(bi, par):
            val_v = vals[par]
            pltpu.make_async_copy(y_slice(bi), val_v.at[pl.ds(0, 2 * nnz)],
                                  sems_in.at[par]).wait()

            @pl.when(bi + 1 < b)
            def _():
                pltpu.async_copy(y_slice(bi + 1),
                                 vals[1 - par].at[pl.ds(0, 2 * nnz)],
                                 sems_in.at[1 - par])

            for ch in range(2):
                if ch == 0:
                    @plsc.parallel_loop(0, nch, unroll=8)
                    def _acc0(i):
                        p = idx_v[pl.ds(i * L, L)]
                        v = val_v[pl.ds(i * L, L)]
                        plsc.addupdate_scatter(acc, [p], v)
                else:
                    @plsc.parallel_loop(0, nch, unroll=8)
                    def _acc1(i):
                        p = idx_v[pl.ds(i * L, L)]
                        v = plsc.load_gather(val_v, [iota16 + (nnz + i * L)])
                        plsc.addupdate_scatter(acc, [p], v)

                @pl.when(bi >= 1)
                def _():
                    pltpu.make_async_copy(
                        obufs[ch], out_slice(bi - 1, ch), sems_out.at[ch]
                    ).wait()

                @plsc.parallel_loop(0, n_out // L, unroll=4)
                def _drain(j):
                    s = acc[pl.ds(j * L, L)]
                    acc[pl.ds(j * L, L)] = zeros16f
                    obufs[ch][pl.ds(j * L, L)] = s

                pltpu.async_copy(obufs[ch], out_slice(bi, ch),
                                 sems_out.at[ch])

        @pl.loop(0, b // 2)
        def _batch(k):
            process(2 * k, 0)
            process(2 * k + 1, 1)

        pltpu.make_async_copy(obuf0, out_slice(b - 1, 0),
                              sems_out.at[0]).wait()
        pltpu.make_async_copy(obuf1, out_slice(b - 1, 1),
                              sems_out.at[1]).wait()

    return scatter_kernel(y_flat, idx0p)


def kernel(features, eval_locs, W0, W1, W2, eval_indices):
    b, c_in, n_in = features.shape
    nnz = eval_indices.shape[0]
    np_pad = ((nnz + TILE - 1) // TILE) * TILE
    nv = ((nnz + L - 1) // L) * L
    n_out = 1024

    idx0 = eval_indices[:, 0].astype(jnp.int32)
    idx1 = eval_indices[:, 1].astype(jnp.int32)
    idx1p = jnp.pad(idx1, (0, nv - nnz))
    idx0p = jnp.pad(idx0, (0, nv - nnz), constant_values=n_out)

    locsT_pad = jnp.pad(eval_locs, ((0, np_pad - nnz), (0, 0))).T

    def tc_stage(x3, bh):
        return pl.pallas_call(
            _mlp_matmul_kernel,
            grid=(np_pad // TILE,),
            in_specs=[
                pl.BlockSpec((2, TILE), lambda i: (0, i)),
                pl.BlockSpec((64, 2), lambda i: (0, 0)),
                pl.BlockSpec((64, 64), lambda i: (0, 0)),
                pl.BlockSpec((C_IN * C_OUT, 64), lambda i: (0, 0)),
                pl.BlockSpec((bh, TILE, C_IN), lambda i: (0, i, 0)),
            ],
            out_specs=pl.BlockSpec((bh, TILE, C_OUT), lambda i: (0, i, 0)),
            out_shape=jax.ShapeDtypeStruct((bh, np_pad, C_OUT), jnp.float32),
        )(locsT_pad, W0.T, W1.T, W2.T, x3)

    bh = b // 2
    outs = []
    for h in range(2):
        feat_h = features[h * bh:(h + 1) * bh].reshape(-1)
        x_flat = _sc_gather(feat_h, idx1p, bh, n_in, nnz, np_pad * C_IN)
        x3 = x_flat.reshape(bh, np_pad, C_IN)
        y3 = tc_stage(x3, bh)
        out_flat = _sc_scatter(y3.reshape(-1), idx0p, bh, nnz,
                               np_pad * C_OUT, n_out)
        outs.append(out_flat.reshape(bh, C_OUT, n_out))
    return jnp.concatenate(outs, axis=0)


# speedup vs baseline: 1.1744x; 1.0067x over previous
import dataclasses
import functools

import jax
import jax.numpy as jnp
from jax import lax
from jax.experimental import pallas as pl
from jax.experimental.pallas import tpu as pltpu
from jax.experimental.pallas import tpu_sc as plsc

C_IN = 64
C_OUT = 64
TILE = 512
L = 16


def _sc_compiler_params():
    cp = pltpu.CompilerParams()
    if "needs_layout_passes" in pltpu.CompilerParams.__dataclass_fields__:
        cp = dataclasses.replace(cp, needs_layout_passes=False)
    return cp


def _mlp_matmul_kernel(locsT_ref, w0t_ref, w1t_ref, w2t_ref, x_ref, y_ref):
    h = jnp.sin(jnp.dot(w0t_ref[...], locsT_ref[...], preferred_element_type=jnp.float32))
    h = jnp.sin(jnp.dot(w1t_ref[...], h, preferred_element_type=jnp.float32))
    gT = jnp.dot(w2t_ref[...].astype(jnp.bfloat16), h.astype(jnp.bfloat16),
                 preferred_element_type=jnp.float32)
    gr = gT.astype(jnp.bfloat16).reshape(C_IN, C_OUT, TILE)
    x = x_ref[...].astype(jnp.bfloat16)
    y = jax.lax.dot_general(
        x, gr,
        dimension_numbers=(((2,), (0,)), ((1,), (2,))),
        preferred_element_type=jnp.float32,
    )
    y_ref[...] = y.transpose(1, 0, 2)


def _sc_gather(feat_flat, idx1p, b, n_in, nnz, np64):
    nv = idx1p.shape[0]
    nch = nv // L
    mesh = plsc.VectorSubcoreMesh(core_axis_name="c", subcore_axis_name="s")

    @functools.partial(
        pl.kernel,
        out_type=jax.ShapeDtypeStruct((b * np64,), jnp.float32),
        mesh=mesh,
        scratch_types=[
            pltpu.VMEM((nv,), jnp.int32),
            pltpu.VMEM((2 * n_in,), jnp.float32),
            pltpu.VMEM((2 * n_in,), jnp.float32),
            pltpu.VMEM((2 * nv,), jnp.float32),
            pltpu.VMEM((2 * nv,), jnp.float32),
            pltpu.SemaphoreType.DMA((4,)),
        ],
        compiler_params=_sc_compiler_params(),
    )
    def gather_kernel(feat_hbm, idx_hbm, x_hbm, idx_v, src0, src1, dst0, dst1,
                      sems):
        w = lax.axis_index("s") * 2 + lax.axis_index("c")
        pltpu.sync_copy(idx_hbm, idx_v)
        iota16 = lax.iota(jnp.int32, L)
        srcs = (src0, src1)
        dsts = (dst0, dst1)

        def feat_slice(bi):
            return feat_hbm.at[pl.ds((bi * C_IN + 2 * w) * n_in, 2 * n_in)]

        def out_slice(bi):
            return x_hbm.at[pl.ds(bi * np64 + w * 2 * nnz, 2 * nnz)]

        pltpu.async_copy(feat_slice(0), src0, sems.at[0])

        def process(bi, par):
            src_v = srcs[par]
            dst_v = dsts[par]
            pltpu.make_async_copy(feat_slice(bi), src_v, sems.at[par]).wait()

            @pl.when(bi + 1 < b)
            def _():
                pltpu.async_copy(feat_slice(bi + 1), srcs[1 - par],
                                 sems.at[1 - par])

            @pl.when(bi >= 2)
            def _():
                pltpu.make_async_copy(
                    dst_v.at[pl.ds(0, 2 * nnz)],
                    out_slice(bi - 2), sems.at[2 + par],
                ).wait()

            @plsc.parallel_loop(0, nch, unroll=4)
            def _ch0(i):
                idx = idx_v[pl.ds(i * L, L)]
                v0 = plsc.load_gather(src_v, [idx])
                dst_v[pl.ds(i * L, L)] = v0

            @plsc.parallel_loop(0, nch, unroll=4)
            def _ch1(i):
                idx = idx_v[pl.ds(i * L, L)]
                v1 = plsc.load_gather(src_v, [idx + n_in])
                plsc.store_scatter(dst_v, [iota16 + (nnz + i * L)], v1)

            pltpu.async_copy(
                dst_v.at[pl.ds(0, 2 * nnz)],
                out_slice(bi), sems.at[2 + par],
            )

        @pl.loop(0, b // 2)
        def _batch(k):
            process(2 * k, 0)
            process(2 * k + 1, 1)

        pltpu.make_async_copy(
            dst0.at[pl.ds(0, 2 * nnz)], out_slice(b - 2), sems.at[2]
        ).wait()
        pltpu.make_async_copy(
            dst1.at[pl.ds(0, 2 * nnz)], out_slice(b - 1), sems.at[3]
        ).wait()

    return gather_kernel(feat_flat, idx1p)


def _sc_scatter(y_flat, idx0p, b, nnz, np64, n_out):
    nv = idx0p.shape[0]
    nch = nv // L
    acc_w = n_out + L
    mesh = plsc.VectorSubcoreMesh(core_axis_name="c", subcore_axis_name="s")

    @functools.partial(
        pl.kernel,
        out_type=jax.ShapeDtypeStruct((b * C_OUT * n_out,), jnp.float32),
        mesh=mesh,
        scratch_types=[
            pltpu.VMEM((nv,), jnp.int32),
            pltpu.VMEM((2 * nv,), jnp.float32),
            pltpu.VMEM((2 * nv,), jnp.float32),
            pltpu.VMEM((acc_w,), jnp.float32),
            pltpu.VMEM((n_out,), jnp.float32),
            pltpu.VMEM((n_out,), jnp.float32),
            pltpu.SemaphoreType.DMA((2,)),
            pltpu.SemaphoreType.DMA((2,)),
        ],
        compiler_params=_sc_compiler_params(),
    )
    def scatter_kernel(y_hbm, idx_hbm, out_hbm, idx_v, val0, val1, acc,
                       obuf0, obuf1, sems_in, sems_out):
        w = lax.axis_index("s") * 2 + lax.axis_index("c")
        pltpu.sync_copy(idx_hbm, idx_v)
        zeros16f = jnp.zeros((L,), jnp.float32)
        iota16 = lax.iota(jnp.int32, L)
        vals = (val0, val1)
        obufs = (obuf0, obuf1)

        def y_slice(bi):
            return y_hbm.at[pl.ds(bi * np64 + w * 2 * nnz, 2 * nnz)]

        def out_slice(bi, ch):
            return out_hbm.at[pl.ds((bi * C_OUT + 2 * w + ch) * n_out, n_out)]

        @plsc.parallel_loop(0, acc_w // L, unroll=4)
        def _zero(z):
            acc[pl.ds(z * L, L)] = zeros16f

        pltpu.async_copy(y_slice(0), val0.at[pl.ds(0, 2 * nnz)], sems_in.at[0])

        def process(bi, par):
            val_v = vals[par]
            pltpu.make_async_copy(y_slice(bi), val_v.at[pl.ds(0, 2 * nnz)],
                                  sems_in.at[par]).wait()

            @pl.when(bi + 1 < b)
            def _():
                pltpu.async_copy(y_slice(bi + 1),
                                 vals[1 - par].at[pl.ds(0, 2 * nnz)],
                                 sems_in.at[1 - par])

            for ch in range(2):
                if ch == 0:
                    @plsc.parallel_loop(0, nch, unroll=4)
                    def _acc0(i):
                        p = idx_v[pl.ds(i * L, L)]
                        v = val_v[pl.ds(i * L, L)]
                        plsc.addupdate_scatter(acc, [p], v)
                else:
                    @plsc.parallel_loop(0, nch, unroll=4)
                    def _acc1(i):
                        p = idx_v[pl.ds(i * L, L)]
                        v = plsc.load_gather(val_v, [iota16 + (nnz + i * L)])
                        plsc.addupdate_scatter(acc, [p], v)

                @pl.when(bi >= 1)
                def _():
                    pltpu.make_async_copy(
                        obufs[ch], out_slice(bi - 1, ch), sems_out.at[ch]
                    ).wait()

                @plsc.parallel_loop(0, n_out // L, unroll=4)
                def _drain(j):
                    s = acc[pl.ds(j * L, L)]
                    acc[pl.ds(j * L, L)] = zeros16f
                    obufs[ch][pl.ds(j * L, L)] = s

                pltpu.async_copy(obufs[ch], out_slice(bi, ch),
                                 sems_out.at[ch])

        @pl.loop(0, b // 2)
        def _batch(k):
            process(2 * k, 0)
            process(2 * k + 1, 1)

        pltpu.make_async_copy(obuf0, out_slice(b - 1, 0),
                              sems_out.at[0]).wait()
        pltpu.make_async_copy(obuf1, out_slice(b - 1, 1),
                              sems_out.at[1]).wait()

    return scatter_kernel(y_flat, idx0p)


def kernel(features, eval_locs, W0, W1, W2, eval_indices):
    b, c_in, n_in = features.shape
    nnz = eval_indices.shape[0]
    np_pad = ((nnz + TILE - 1) // TILE) * TILE
    nv = ((nnz + L - 1) // L) * L
    n_out = 1024

    idx0 = eval_indices[:, 0].astype(jnp.int32)
    idx1 = eval_indices[:, 1].astype(jnp.int32)
    idx1p = jnp.pad(idx1, (0, nv - nnz))
    idx0p = jnp.pad(idx0, (0, nv - nnz), constant_values=n_out)

    locsT_pad = jnp.pad(eval_locs, ((0, np_pad - nnz), (0, 0))).T

    def tc_stage(x3, bh):
        return pl.pallas_call(
            _mlp_matmul_kernel,
            grid=(np_pad // TILE,),
            in_specs=[
                pl.BlockSpec((2, TILE), lambda i: (0, i)),
                pl.BlockSpec((64, 2), lambda i: (0, 0)),
                pl.BlockSpec((64, 64), lambda i: (0, 0)),
                pl.BlockSpec((C_IN * C_OUT, 64), lambda i: (0, 0)),
                pl.BlockSpec((bh, TILE, C_IN), lambda i: (0, i, 0)),
            ],
            out_specs=pl.BlockSpec((bh, TILE, C_OUT), lambda i: (0, i, 0)),
            out_shape=jax.ShapeDtypeStruct((bh, np_pad, C_OUT), jnp.float32),
        )(locsT_pad, W0.T, W1.T, W2.T, x3)

    bh = b // 2
    outs = []
    for h in range(2):
        feat_h = features[h * bh:(h + 1) * bh].reshape(-1)
        x_flat = _sc_gather(feat_h, idx1p, bh, n_in, nnz, np_pad * C_IN)
        x3 = x_flat.reshape(bh, np_pad, C_IN)
        y3 = tc_stage(x3, bh)
        out_flat = _sc_scatter(y3.reshape(-1), idx0p, bh, nnz,
                               np_pad * C_OUT, n_out)
        outs.append(out_flat.reshape(bh, C_OUT, n_out))
    return jnp.concatenate(outs, axis=0)
